# baseline (speedup 1.0000x reference)
import functools

import jax
import jax.numpy as jnp
from jax import lax
from jax.experimental import pallas as pl
from jax.experimental.pallas import tpu as pltpu, tpu_sc as plsc

N_ROWS = 100000
DIM = 64
NW = 32
CHUNK_ROWS = 800
CHUNK_ELEMS = CHUNK_ROWS * DIM
GROUPS = CHUNK_ROWS // 16
N_CHUNKS = N_ROWS // CHUNK_ROWS
NEG_BIG = -3e38


@functools.partial(
    pl.kernel,
    out_type=jax.ShapeDtypeStruct((NW, 4 * 16), jnp.float32),
    mesh=plsc.VectorSubcoreMesh(core_axis_name="c", subcore_axis_name="s"),
    compiler_params=pltpu.CompilerParams(needs_layout_passes=False),
    scratch_types=[
        pltpu.VMEM((CHUNK_ELEMS,), jnp.float32),
        pltpu.VMEM((CHUNK_ROWS,), jnp.int32),
        pltpu.VMEM((DIM,), jnp.float32),
        pltpu.VMEM((16,), jnp.float32),
        pltpu.VMEM((4 * 16,), jnp.float32),
    ],
)
def _sc_partials(e_ref, y_ref, p_ref, aux_ref, out_ref,
                 rows_v, y_v, p_v, aux_v, stage_v):
    w = lax.axis_index("s") * 2 + lax.axis_index("c")
    pltpu.sync_copy(p_ref, p_v)
    pltpu.sync_copy(aux_ref, aux_v)
    aux16 = aux_v[pl.ds(0, 16)]
    denom_scale = aux16[0]
    denom_floor = aux16[1]
    pvecs = tuple(p_v[pl.ds(16 * i, 16)] for i in range(DIM // 16))
    proto = [pvecs[k // 16][k % 16] for k in range(DIM)]
    lane_base = lax.iota(jnp.int32, 16) * DIM

    def chunk_body(j, carry):
        cid = w + NW * j
        pltpu.sync_copy(e_ref.at[pl.ds(cid * CHUNK_ELEMS, CHUNK_ELEMS)], rows_v)
        pltpu.sync_copy(y_ref.at[pl.ds(cid * CHUNK_ROWS, CHUNK_ROWS)], y_v)

        def group_body(g, acc):
            m, s, t, c = acc
            mask = y_v[pl.ds(g * 16, 16)] == 0
            idx = g * (16 * DIM) + lane_base
            dot = jnp.zeros((16,), jnp.float32)
            sq = jnp.zeros((16,), jnp.float32)
            for k in range(DIM):
                v = plsc.load_gather(rows_v, [idx + k])
                dot = dot + v * proto[k]
                sq = sq + v * v
            bits = plsc.bitcast(sq, jnp.int32)
            r = plsc.bitcast(jnp.int32(0x5F3759DF) - (bits >> 1), jnp.float32)
            r = r * (1.5 - 0.5 * sq * r * r)
            r = r * (1.5 - 0.5 * sq * r * r)
            r = r * (1.5 - 0.5 * sq * r * r)
            norm = sq * r
            scaled = dot / jnp.maximum(norm * denom_scale, denom_floor)
            xm = jnp.where(mask, -scaled, NEG_BIG)
            new_m = jnp.maximum(m, xm)
            s = s * jnp.exp(m - new_m) + jnp.where(mask, jnp.exp(xm - new_m), 0.0)
            t = t + jnp.where(mask, scaled, 0.0)
            c = c + jnp.where(mask, 1.0, 0.0)
            return (new_m, s, t, c)

        return lax.fori_loop(0, GROUPS, group_body, carry)

    init = (jnp.full((16,), NEG_BIG, jnp.float32),
            jnp.zeros((16,), jnp.float32),
            jnp.zeros((16,), jnp.float32),
            jnp.zeros((16,), jnp.float32))
    n_chunks_w = jnp.where(w < N_CHUNKS - 3 * NW, 4, 3)
    m, s, t, c = lax.fori_loop(0, n_chunks_w, chunk_body, init)
    stage_v[pl.ds(0, 16)] = m
    stage_v[pl.ds(16, 16)] = s
    stage_v[pl.ds(32, 16)] = t
    stage_v[pl.ds(48, 16)] = c
    pltpu.sync_copy(stage_v, out_ref.at[w])


def kernel(class_embeddings, support_y, negative_prototypes, temperature):
    e_flat = class_embeddings.reshape(-1)
    y = support_y.astype(jnp.int32)
    p = negative_prototypes.astype(jnp.float32)
    t = jnp.asarray(temperature, jnp.float32)
    nb = jnp.sqrt(jnp.sum(p * p))
    aux = jnp.zeros((16,), jnp.float32).at[0].set(nb * t).at[1].set(1e-8 * t)
    parts = _sc_partials(e_flat, y, p, aux)
    m = parts[:, 0:16].reshape(-1)
    s = parts[:, 16:32].reshape(-1)
    tsum = parts[:, 32:48].reshape(-1)
    cnt = parts[:, 48:64].reshape(-1)
    mg = jnp.max(m)
    sg = jnp.sum(s * jnp.exp(m - mg))
    lse = mg + jnp.log(sg)
    return lse + jnp.sum(tsum) / jnp.sum(cnt)


# speedup vs baseline: 1.0502x; 1.0502x over previous
import functools

import jax
import jax.numpy as jnp
from jax import lax
from jax.experimental import pallas as pl
from jax.experimental.pallas import tpu as pltpu, tpu_sc as plsc

N_ROWS = 100000
DIM = 64
NW = 32
CHUNK_ROWS = 800
CHUNK_ELEMS = CHUNK_ROWS * DIM
GROUPS = CHUNK_ROWS // 16
N_CHUNKS = N_ROWS // CHUNK_ROWS
MAX_CHUNKS_PER_W = -(-N_CHUNKS // NW)
GATHER_SPAN = 15 * DIM + 8
NEG_BIG = -3e38


@functools.partial(
    pl.kernel,
    out_type=jax.ShapeDtypeStruct((NW, 4 * 16), jnp.float32),
    mesh=plsc.VectorSubcoreMesh(core_axis_name="c", subcore_axis_name="s"),
    compiler_params=pltpu.CompilerParams(needs_layout_passes=False),
    scratch_types=[
        pltpu.VMEM((CHUNK_ELEMS,), jnp.float32),
        pltpu.VMEM((CHUNK_ELEMS,), jnp.float32),
        pltpu.VMEM((CHUNK_ROWS,), jnp.int32),
        pltpu.VMEM((CHUNK_ROWS,), jnp.int32),
        pltpu.VMEM((DIM,), jnp.float32),
        pltpu.VMEM((16,), jnp.float32),
        pltpu.SMEM((DIM,), jnp.float32),
        pltpu.VMEM((4 * 16,), jnp.float32),
        pltpu.SemaphoreType.DMA,
        pltpu.SemaphoreType.DMA,
    ],
)
def _sc_partials(e_ref, y_ref, p_ref, aux_ref, out_ref,
                 rows_a, rows_b, y_a, y_b, p_v, aux_v, p_s, stage_v,
                 sem_a, sem_b):
    w = lax.axis_index("s") * 2 + lax.axis_index("c")
    pltpu.sync_copy(p_ref, p_v)
    pltpu.sync_copy(aux_ref, aux_v)
    aux16 = aux_v[...]
    denom_scale = aux16[0]
    denom_floor = aux16[1]
    for i in range(DIM // 16):
        pv = p_v[pl.ds(16 * i, 16)]
        for l in range(16):
            p_s[16 * i + l] = pv[l]
    lane_base = lax.iota(jnp.int32, 16) * DIM
    lane_idx = tuple(lane_base + d for d in range(8))

    rows_bufs = (rows_a, rows_b)
    y_bufs = (y_a, y_b)
    sems = (sem_a, sem_b)

    def issue(j, buf):
        cid = jnp.minimum(w + NW * j, N_CHUNKS - 1)
        c1 = pltpu.async_copy(
            e_ref.at[pl.ds(cid * CHUNK_ELEMS, CHUNK_ELEMS)], rows_bufs[buf],
            sems[buf])
        c2 = pltpu.async_copy(
            y_ref.at[pl.ds(cid * CHUNK_ROWS, CHUNK_ROWS)], y_bufs[buf],
            sems[buf])
        return c1, c2

    pending = issue(0, 0)

    state = (jnp.full((16,), NEG_BIG, jnp.float32),
             jnp.zeros((16,), jnp.float32),
             jnp.zeros((16,), jnp.float32),
             jnp.zeros((16,), jnp.float32))

    for j in range(MAX_CHUNKS_PER_W):
        buf = j % 2
        for cpy in pending:
            cpy.wait()
        if j + 1 < MAX_CHUNKS_PER_W:
            pending = issue(j + 1, buf ^ 1)
        rows_v = rows_bufs[buf]
        y_v = y_bufs[buf]
        valid = w + NW * j < N_CHUNKS

        def group_body(g, acc, rows_v=rows_v, y_v=y_v, valid=valid):
            m, s, t, c = acc
            mask = (y_v[pl.ds(g * 16, 16)] == 0) & valid
            gbase = g * (16 * DIM)
            dot = jnp.zeros((16,), jnp.float32)
            sq = jnp.zeros((16,), jnp.float32)
            for k in range(DIM):
                v = plsc.load_gather(
                    rows_v.at[pl.ds(gbase + (k & ~7), GATHER_SPAN)],
                    [lane_idx[k & 7]])
                dot = dot + v * p_s[k]
                sq = sq + v * v
            bits = plsc.bitcast(sq, jnp.int32)
            r = plsc.bitcast(jnp.int32(0x5F3759DF) - (bits >> 1), jnp.float32)
            r = r * (1.5 - 0.5 * sq * r * r)
            r = r * (1.5 - 0.5 * sq * r * r)
            r = r * (1.5 - 0.5 * sq * r * r)
            norm = sq * r
            scaled = dot / jnp.maximum(norm * denom_scale, denom_floor)
            xm = jnp.where(mask, -scaled, NEG_BIG)
            new_m = jnp.maximum(m, xm)
            s = s * jnp.exp(m - new_m) + jnp.where(mask, jnp.exp(xm - new_m), 0.0)
            t = t + jnp.where(mask, scaled, 0.0)
            c = c + jnp.where(mask, 1.0, 0.0)
            return (new_m, s, t, c)

        state = lax.fori_loop(0, GROUPS, group_body, state)

    m, s, t, c = state
    stage_v[pl.ds(0, 16)] = m
    stage_v[pl.ds(16, 16)] = s
    stage_v[pl.ds(32, 16)] = t
    stage_v[pl.ds(48, 16)] = c
    pltpu.sync_copy(stage_v, out_ref.at[w])


def kernel(class_embeddings, support_y, negative_prototypes, temperature):
    e_flat = class_embeddings.reshape(-1)
    y = support_y.astype(jnp.int32)
    p = negative_prototypes.astype(jnp.float32)
    t = jnp.asarray(temperature, jnp.float32)
    nb = jnp.sqrt(jnp.sum(p * p))
    aux = jnp.zeros((16,), jnp.float32).at[0].set(nb * t).at[1].set(1e-8 * t)
    parts = _sc_partials(e_flat, y, p, aux)
    m = parts[:, 0:16].reshape(-1)
    s = parts[:, 16:32].reshape(-1)
    tsum = parts[:, 32:48].reshape(-1)
    cnt = parts[:, 48:64].reshape(-1)
    mg = jnp.max(m)
    sg = jnp.sum(s * jnp.exp(m - mg))
    lse = mg + jnp.log(sg)
    return lse + jnp.sum(tsum) / jnp.sum(cnt)


# speedup vs baseline: 1.5701x; 1.4950x over previous
import functools

import jax
import jax.numpy as jnp
from jax import lax
from jax.experimental import pallas as pl
from jax.experimental.pallas import tpu as pltpu, tpu_sc as plsc

N_ROWS = 100000
DIM = 64
NW = 32
CHUNK_ROWS = 800
CHUNK_ELEMS = CHUNK_ROWS * DIM
GROUPS = CHUNK_ROWS // 16
N_CHUNKS = N_ROWS // CHUNK_ROWS
MAX_CHUNKS_PER_W = -(-N_CHUNKS // NW)
GATHER_SPAN = 15 * DIM + 16
NEG_BIG = -3e38


@functools.partial(
    pl.kernel,
    out_type=jax.ShapeDtypeStruct((NW, 4 * 16), jnp.float32),
    mesh=plsc.VectorSubcoreMesh(core_axis_name="c", subcore_axis_name="s"),
    compiler_params=pltpu.CompilerParams(needs_layout_passes=False),
    scratch_types=[
        pltpu.VMEM((CHUNK_ELEMS,), jnp.float32),
        pltpu.VMEM((CHUNK_ELEMS,), jnp.float32),
        pltpu.VMEM((CHUNK_ROWS,), jnp.int32),
        pltpu.VMEM((CHUNK_ROWS,), jnp.int32),
        pltpu.VMEM((DIM,), jnp.float32),
        pltpu.VMEM((16,), jnp.float32),
        pltpu.VMEM((4 * 16,), jnp.float32),
        pltpu.SemaphoreType.DMA,
        pltpu.SemaphoreType.DMA,
    ],
)
def _sc_partials(e_ref, y_ref, p_ref, aux_ref, out_ref,
                 rows_a, rows_b, y_a, y_b, p_v, aux_v, stage_v,
                 sem_a, sem_b):
    w = lax.axis_index("s") * 2 + lax.axis_index("c")
    pltpu.sync_copy(p_ref, p_v)
    pltpu.sync_copy(aux_ref, aux_v)
    aux16 = aux_v[...]
    denom_scale = aux16[0]
    denom_floor = aux16[1]
    pvecs = tuple(p_v[pl.ds(16 * i, 16)] for i in range(DIM // 16))
    lanes = lax.iota(jnp.int32, 16)
    rot_sel = tuple((lanes + k16) & 15 for k16 in range(16))
    lane_idx = tuple(lanes * DIM + rot_sel[k16] for k16 in range(16))

    rows_bufs = (rows_a, rows_b)
    y_bufs = (y_a, y_b)
    sems = (sem_a, sem_b)

    def issue(j, buf):
        cid = jnp.minimum(w + NW * j, N_CHUNKS - 1)
        c1 = pltpu.async_copy(
            e_ref.at[pl.ds(cid * CHUNK_ELEMS, CHUNK_ELEMS)], rows_bufs[buf],
            sems[buf])
        c2 = pltpu.async_copy(
            y_ref.at[pl.ds(cid * CHUNK_ROWS, CHUNK_ROWS)], y_bufs[buf],
            sems[buf])
        return c1, c2

    pending = issue(0, 0)

    state = (jnp.full((16,), NEG_BIG, jnp.float32),
             jnp.zeros((16,), jnp.float32),
             jnp.zeros((16,), jnp.float32),
             jnp.zeros((16,), jnp.float32))

    for j in range(MAX_CHUNKS_PER_W):
        buf = j % 2
        for cpy in pending:
            cpy.wait()
        if j + 1 < MAX_CHUNKS_PER_W:
            pending = issue(j + 1, buf ^ 1)
        rows_v = rows_bufs[buf]
        y_v = y_bufs[buf]
        valid = w + NW * j < N_CHUNKS

        def group_body(g, acc, rows_v=rows_v, y_v=y_v, valid=valid):
            m, s, t, c = acc
            mask = (y_v[pl.ds(g * 16, 16)] == 0) & valid
            gbase = g * (16 * DIM)
            dot = jnp.zeros((16,), jnp.float32)
            sq = jnp.zeros((16,), jnp.float32)
            for k in range(DIM):
                blk, k16 = divmod(k, 16)
                v = plsc.load_gather(
                    rows_v.at[pl.ds(gbase + 16 * blk, GATHER_SPAN)],
                    [lane_idx[k16]])
                rot = pvecs[blk].at[rot_sel[k16]].get(
                    mode="promise_in_bounds", unique_indices=True)
                dot = dot + v * rot
                sq = sq + v * v
            bits = plsc.bitcast(sq, jnp.int32)
            r = plsc.bitcast(jnp.int32(0x5F3759DF) - (bits >> 1), jnp.float32)
            r = r * (1.5 - 0.5 * sq * r * r)
            r = r * (1.5 - 0.5 * sq * r * r)
            r = r * (1.5 - 0.5 * sq * r * r)
            norm = sq * r
            scaled = dot / jnp.maximum(norm * denom_scale, denom_floor)
            xm = jnp.where(mask, -scaled, NEG_BIG)
            new_m = jnp.maximum(m, xm)
            s = s * jnp.exp(m - new_m) + jnp.where(mask, jnp.exp(xm - new_m), 0.0)
            t = t + jnp.where(mask, scaled, 0.0)
            c = c + jnp.where(mask, 1.0, 0.0)
            return (new_m, s, t, c)

        state = lax.fori_loop(0, GROUPS, group_body, state)

    m, s, t, c = state
    stage_v[pl.ds(0, 16)] = m
    stage_v[pl.ds(16, 16)] = s
    stage_v[pl.ds(32, 16)] = t
    stage_v[pl.ds(48, 16)] = c
    pltpu.sync_copy(stage_v, out_ref.at[w])


def kernel(class_embeddings, support_y, negative_prototypes, temperature):
    e_flat = class_embeddings.reshape(-1)
    y = support_y.astype(jnp.int32)
    p = negative_prototypes.astype(jnp.float32)
    t = jnp.asarray(temperature, jnp.float32)
    nb = jnp.sqrt(jnp.sum(p * p))
    aux = jnp.zeros((16,), jnp.float32).at[0].set(nb * t).at[1].set(1e-8 * t)
    parts = _sc_partials(e_flat, y, p, aux)
    m = parts[:, 0:16].reshape(-1)
    s = parts[:, 16:32].reshape(-1)
    tsum = parts[:, 32:48].reshape(-1)
    cnt = parts[:, 48:64].reshape(-1)
    mg = jnp.max(m)
    sg = jnp.sum(s * jnp.exp(m - mg))
    lse = mg + jnp.log(sg)
    return lse + jnp.sum(tsum) / jnp.sum(cnt)


# speedup vs baseline: 1.9258x; 1.2266x over previous
import functools

import jax
import jax.numpy as jnp
from jax import lax
from jax.experimental import pallas as pl
from jax.experimental.pallas import tpu as pltpu, tpu_sc as plsc

N_ROWS = 100000
DIM = 64
NW = 32
CHUNK_ROWS = 800
CHUNK_ELEMS = CHUNK_ROWS * DIM
GROUPS = CHUNK_ROWS // 16
N_CHUNKS = N_ROWS // CHUNK_ROWS
MAX_CHUNKS_PER_W = -(-N_CHUNKS // NW)
GATHER_SPAN = 15 * DIM + 16
NEG_BIG = -3e38


@functools.partial(
    pl.kernel,
    out_type=jax.ShapeDtypeStruct((NW, 4 * 16), jnp.float32),
    mesh=plsc.VectorSubcoreMesh(core_axis_name="c", subcore_axis_name="s"),
    compiler_params=pltpu.CompilerParams(needs_layout_passes=False),
    scratch_types=[
        pltpu.VMEM((CHUNK_ELEMS,), jnp.float32),
        pltpu.VMEM((CHUNK_ELEMS,), jnp.float32),
        pltpu.VMEM((CHUNK_ROWS,), jnp.int32),
        pltpu.VMEM((CHUNK_ROWS,), jnp.int32),
        pltpu.VMEM((DIM * 16,), jnp.float32),
        pltpu.VMEM((16,), jnp.float32),
        pltpu.VMEM((4 * 16,), jnp.float32),
        pltpu.SemaphoreType.DMA,
        pltpu.SemaphoreType.DMA,
    ],
)
def _sc_partials(e_ref, y_ref, p_ref, aux_ref, out_ref,
                 rows_a, rows_b, y_a, y_b, p_v, aux_v, stage_v,
                 sem_a, sem_b):
    w = lax.axis_index("s") * 2 + lax.axis_index("c")
    pltpu.sync_copy(p_ref, p_v)
    pltpu.sync_copy(aux_ref, aux_v)
    aux16 = aux_v[...]
    denom_scale = aux16[0]
    denom_floor = aux16[1]
    lanes = lax.iota(jnp.int32, 16)
    rot_sel = tuple((lanes + k16) & 15 for k16 in range(16))
    lane_idx = tuple(lanes * DIM + rot_sel[k16] for k16 in range(16))

    rows_bufs = (rows_a, rows_b)
    y_bufs = (y_a, y_b)
    sems = (sem_a, sem_b)

    def issue(j, buf):
        cid = jnp.minimum(w + NW * j, N_CHUNKS - 1)
        c1 = pltpu.async_copy(
            e_ref.at[pl.ds(cid * CHUNK_ELEMS, CHUNK_ELEMS)], rows_bufs[buf],
            sems[buf])
        c2 = pltpu.async_copy(
            y_ref.at[pl.ds(cid * CHUNK_ROWS, CHUNK_ROWS)], y_bufs[buf],
            sems[buf])
        return c1, c2

    pending = issue(0, 0)

    state = (jnp.full((16,), NEG_BIG, jnp.float32),
             jnp.zeros((16,), jnp.float32),
             jnp.zeros((16,), jnp.float32),
             jnp.zeros((16,), jnp.float32))

    for j in range(MAX_CHUNKS_PER_W):
        buf = j % 2
        for cpy in pending:
            cpy.wait()
        if j + 1 < MAX_CHUNKS_PER_W:
            pending = issue(j + 1, buf ^ 1)
        rows_v = rows_bufs[buf]
        y_v = y_bufs[buf]
        valid = w + NW * j < N_CHUNKS

        def group_body(g, acc, rows_v=rows_v, y_v=y_v, valid=valid):
            m, s, t, c = acc
            mask = (y_v[pl.ds(g * 16, 16)] == 0) & valid
            gbase = g * (16 * DIM)
            dot = jnp.zeros((16,), jnp.float32)
            sq = jnp.zeros((16,), jnp.float32)
            for k in range(DIM):
                blk, k16 = divmod(k, 16)
                v = plsc.load_gather(
                    rows_v.at[pl.ds(gbase + 16 * blk, GATHER_SPAN)],
                    [lane_idx[k16]])
                rot = p_v[pl.ds(16 * k, 16)]
                dot = dot + v * rot
                sq = sq + v * v
            bits = plsc.bitcast(sq, jnp.int32)
            r = plsc.bitcast(jnp.int32(0x5F3759DF) - (bits >> 1), jnp.float32)
            r = r * (1.5 - 0.5 * sq * r * r)
            r = r * (1.5 - 0.5 * sq * r * r)
            r = r * (1.5 - 0.5 * sq * r * r)
            norm = sq * r
            scaled = dot / jnp.maximum(norm * denom_scale, denom_floor)
            xm = jnp.where(mask, -scaled, NEG_BIG)
            new_m = jnp.maximum(m, xm)
            s = s * jnp.exp(m - new_m) + jnp.where(mask, jnp.exp(xm - new_m), 0.0)
            t = t + jnp.where(mask, scaled, 0.0)
            c = c + jnp.where(mask, 1.0, 0.0)
            return (new_m, s, t, c)

        state = lax.fori_loop(0, GROUPS, group_body, state)

    m, s, t, c = state
    stage_v[pl.ds(0, 16)] = m
    stage_v[pl.ds(16, 16)] = s
    stage_v[pl.ds(32, 16)] = t
    stage_v[pl.ds(48, 16)] = c
    pltpu.sync_copy(stage_v, out_ref.at[w])


def kernel(class_embeddings, support_y, negative_prototypes, temperature):
    e_flat = class_embeddings.reshape(-1)
    y = support_y.astype(jnp.int32)
    p = negative_prototypes.astype(jnp.float32)
    t = jnp.asarray(temperature, jnp.float32)
    nb = jnp.sqrt(jnp.sum(p * p))
    aux = jnp.zeros((16,), jnp.float32).at[0].set(nb * t).at[1].set(1e-8 * t)
    kk = jnp.arange(DIM)[:, None]
    ll = jnp.arange(16)[None, :]
    rot_table = p[16 * (kk // 16) + (kk % 16 + ll) % 16].reshape(-1)
    parts = _sc_partials(e_flat, y, rot_table, aux)
    m = parts[:, 0:16].reshape(-1)
    s = parts[:, 16:32].reshape(-1)
    tsum = parts[:, 32:48].reshape(-1)
    cnt = parts[:, 48:64].reshape(-1)
    mg = jnp.max(m)
    sg = jnp.sum(s * jnp.exp(m - mg))
    lse = mg + jnp.log(sg)
    return lse + jnp.sum(tsum) / jnp.sum(cnt)
